# baseline (speedup 1.0000x reference)
import functools

import jax
import jax.numpy as jnp
import numpy as np
from jax.experimental import pallas as pl

E = 320000
N = 10000
RATIO = 0.8
K_STATIC = int(np.ceil(RATIO * E))


def _score_conv_kernel(xs_ref, xd_ref, ea_ref, w_ref, out_ref):
    e_blk = jnp.concatenate(
        [xs_ref[...], xd_ref[...], ea_ref[...]], axis=1)
    out_ref[...] = jax.lax.dot_general(
        e_blk, w_ref[...], (((1,), (0,)), ((), ())),
        preferred_element_type=jnp.float32)


def _score_conv(xs, xd, eab, w2):
    blk = 8000
    return pl.pallas_call(
        _score_conv_kernel,
        grid=(E // blk,),
        in_specs=[
            pl.BlockSpec((blk, 128), lambda i: (i, 0)),
            pl.BlockSpec((blk, 128), lambda i: (i, 0)),
            pl.BlockSpec((blk, 16), lambda i: (i, 0)),
            pl.BlockSpec((272, 2), lambda i: (0, 0)),
        ],
        out_specs=pl.BlockSpec((blk, 2), lambda i: (i, 0)),
        out_shape=jax.ShapeDtypeStruct((E, 2), jnp.float32),
    )(xs, xd, eab, w2)


def kernel(x, edge_index, edge_attr, batch, Wf, bf, Ws, bs):
    src = edge_index[0]
    dst = edge_index[1]
    xb = x.astype(jnp.bfloat16)
    eab = edge_attr.astype(jnp.bfloat16)
    xpad = jnp.concatenate(
        [xb, jnp.zeros((140000 - N, 128), jnp.bfloat16)], axis=0)
    xs = xpad[src]
    xd = xpad[dst]
    lfls = _score_conv(xs, xd, eab, jnp.concatenate([Wf, Ws], axis=1))
    raw = jax.nn.sigmoid(lfls[:, 0] + bf[0]) * jax.nn.softplus(lfls[:, 1] + bs[0])
    kint = jax.lax.bitcast_convert_type(raw, jnp.int32)
    perm = jnp.argsort(-kint)[:K_STATIC]
    edge_score = raw[perm][:, None]
    edge_attr_out = edge_attr[perm]
    edge_index_out = jnp.stack([src[perm], dst[perm]])
    return (edge_index_out, edge_attr_out, edge_score)


# speedup vs baseline: 1.9025x; 1.9025x over previous
import functools

import jax
import jax.numpy as jnp
import numpy as np
from jax.experimental import pallas as pl

E = 320000
N = 10000
RATIO = 0.8
K_STATIC = int(np.ceil(RATIO * E))


def _score_conv_kernel(xs_ref, xd_ref, ea_ref, w_ref, out_ref):
    e_blk = jnp.concatenate(
        [xs_ref[...], xd_ref[...], ea_ref[...]], axis=1)
    out_ref[...] = jax.lax.dot_general(
        e_blk, w_ref[...], (((1,), (0,)), ((), ())),
        preferred_element_type=jnp.float32)


def _score_conv(xs, xd, eab, w2):
    blk = 8000
    return pl.pallas_call(
        _score_conv_kernel,
        grid=(E // blk,),
        in_specs=[
            pl.BlockSpec((blk, 128), lambda i: (i, 0)),
            pl.BlockSpec((blk, 128), lambda i: (i, 0)),
            pl.BlockSpec((blk, 16), lambda i: (i, 0)),
            pl.BlockSpec((272, 2), lambda i: (0, 0)),
        ],
        out_specs=pl.BlockSpec((blk, 2), lambda i: (i, 0)),
        out_shape=jax.ShapeDtypeStruct((E, 2), jnp.float32),
    )(xs, xd, eab, w2)


def kernel(x, edge_index, edge_attr, batch, Wf, bf, Ws, bs):
    src = edge_index[0]
    dst = edge_index[1]
    xb = x.astype(jnp.bfloat16)
    eab = edge_attr.astype(jnp.bfloat16)
    xs = xb[src]
    xd = xb[dst]
    lfls = _score_conv(xs, xd, eab, jnp.concatenate([Wf, Ws], axis=1))
    raw = jax.nn.sigmoid(lfls[:, 0] + bf[0]) * jax.nn.softplus(lfls[:, 1] + bs[0])
    kint = jax.lax.bitcast_convert_type(raw, jnp.int32)
    perm = jnp.argsort(-kint)[:K_STATIC]
    edge_score = raw[perm][:, None]
    edge_attr_out = edge_attr[perm]
    edge_index_out = edge_index[:, perm]
    return (edge_index_out, edge_attr_out, edge_score)
